# baseline (speedup 1.0000x reference)
import functools

import jax
import jax.numpy as jnp
from jax import lax
from jax.experimental import pallas as pl
from jax.experimental.pallas import tpu as pltpu
from jax.experimental.pallas import tpu_sc as plsc

DIM = 64
BCOL = 512
SLOT = 96
EPS2 = 1e-24


def _rsqrt(x):
    i = plsc.bitcast(x, jnp.int32)
    i = jnp.int32(0x5F3759DF) - lax.shift_right_arithmetic(i, 1)
    y = plsc.bitcast(i, jnp.float32)
    for _ in range(3):
        y = y * (1.5 - 0.5 * x * y * y)
    return y


def _tree_sum(vs):
    while len(vs) > 1:
        vs = [a + b for a, b in zip(vs[::2], vs[1::2])]
    return vs[0]


@functools.lru_cache(maxsize=None)
def _make_extract_kernel(n_ents_p1: int, n_req: int):
    info = plsc.get_sparse_core_info()
    nw = info.num_cores * info.num_subcores
    nl = info.num_lanes
    nblk = -(-n_ents_p1 // BCOL)
    lastw = n_ents_p1 - (nblk - 1) * BCOL
    base = nblk // nw
    extra = nblk - base * nw
    nbw = base + (1 if extra else 0)
    assert nbw % 2 == 0
    win = 4096
    nbounds = nblk + 1 + 21

    mesh = plsc.VectorSubcoreMesh(core_axis_name="c", subcore_axis_name="s")

    @functools.partial(
        pl.kernel,
        mesh=mesh,
        out_type=jax.ShapeDtypeStruct((nw * nbw * SLOT * DIM,), jnp.float32),
        compiler_params=pltpu.CompilerParams(needs_layout_passes=False),
        scratch_types=[
            pltpu.VMEM((nbounds,), jnp.int32),
            pltpu.VMEM((win + nl,), jnp.int32),
            pltpu.VMEM((DIM, BCOL), jnp.float32),
            pltpu.VMEM((DIM, BCOL), jnp.float32),
            pltpu.VMEM((DIM, lastw), jnp.float32),
            pltpu.VMEM((SLOT * DIM,), jnp.float32),
            pltpu.VMEM((SLOT * DIM,), jnp.float32),
            pltpu.SemaphoreType.DMA,
            pltpu.SemaphoreType.DMA,
            pltpu.SemaphoreType.DMA,
            pltpu.SemaphoreType.DMA,
        ],
    )
    def extract_kernel(entst_hbm, tail_hbm, sidx_hbm, bounds_hbm, gout_hbm,
                       bounds_v, sidx_v, blka, blkb, tailv, stga, stgb,
                       ssa, ssb, swa, swb):
        wid = lax.axis_index("s") * info.num_cores + lax.axis_index("c")
        myblocks = base + jnp.where(wid < extra, 1, 0)
        first = base * wid + jnp.minimum(wid, extra)
        pltpu.sync_copy(bounds_hbm, bounds_v)
        vf = bounds_v[pl.ds(first, nl)]
        rs0 = vf[0]
        rwin = pl.multiple_of(jnp.minimum(rs0, n_req - win) & jnp.int32(~7), 8)
        pltpu.sync_copy(sidx_hbm.at[pl.ds(rwin, win)],
                        sidx_v.at[pl.ds(0, win)])
        pltpu.sync_copy(tail_hbm, tailv)
        lanes = lax.iota(jnp.int32, nl)
        lanes_d = lanes * DIM
        gbase = wid * (nbw * SLOT * DIM)

        def fire(t, blk, ss):
            b = first + t

            @pl.when(b < nblk - 1)
            def _():
                pltpu.async_copy(
                    entst_hbm.at[:, pl.ds(b * BCOL, BCOL)], blk, ss)

        def drain_stream(t, blk, ss):
            b = first + t

            @pl.when(b < nblk - 1)
            def _():
                pltpu.make_async_copy(
                    entst_hbm.at[:, pl.ds(0, BCOL)], blk, ss).wait()

        def drain_write(stg, sw):
            pltpu.make_async_copy(
                gout_hbm.at[pl.ds(0, SLOT * DIM)], stg, sw).wait()

        def process(t, blk, stg, sw):
            b = first + t
            col0 = b * BCOL
            vb = bounds_v[pl.ds(b, nl)]
            b0, b1 = vb[0], vb[1]
            ngroups = lax.shift_right_logical(b1 - b0 + (nl - 1), 4)
            zi = jnp.zeros((nl,), jnp.int32)

            def grp_from(src, width):
                def grp(gi, carry):
                    widx = b0 - rwin + gi * nl
                    ev = sidx_v[pl.ds(widx, nl)]
                    cols = jnp.clip(ev - col0, 0, width - 1)
                    for j in range(DIM):
                        g = plsc.load_gather(src, [zi + j, cols])
                        plsc.store_scatter(
                            stg, [lanes_d + (gi * nl * DIM + j)], g)
                    return carry
                return grp

            @pl.when(b < nblk - 1)
            def _():
                lax.fori_loop(0, ngroups, grp_from(blk, BCOL), 0)

            @pl.when(b == nblk - 1)
            def _():
                lax.fori_loop(0, ngroups, grp_from(tailv, lastw), 0)

            pltpu.async_copy(
                stg, gout_hbm.at[pl.ds(gbase + t * (SLOT * DIM),
                                       SLOT * DIM)], sw)

        @pl.when(0 < myblocks)
        def _():
            fire(0, blka, ssa)

        def pair_body(t2, carry):
            be = 2 * t2
            bo = be + 1

            @pl.when(bo < myblocks)
            def _():
                fire(bo, blkb, ssb)

            @pl.when(be < myblocks)
            def _():
                @pl.when(be >= 2)
                def _():
                    drain_write(stga, swa)

                drain_stream(be, blka, ssa)
                process(be, blka, stga, swa)

            @pl.when(bo + 1 < myblocks)
            def _():
                fire(bo + 1, blka, ssa)

            @pl.when(bo < myblocks)
            def _():
                @pl.when(bo >= 2)
                def _():
                    drain_write(stgb, swb)

                drain_stream(bo, blkb, ssb)
                process(bo, blkb, stgb, swb)

            return carry

        lax.fori_loop(0, nbw // 2, pair_body, 0)
        drain_write(stga, swa)
        drain_write(stgb, swb)

    return extract_kernel, nw, nbw, nblk, base, extra


@functools.lru_cache(maxsize=None)
def _make_score_kernel(n_total: int, n_gout: int, chunk: int):
    info = plsc.get_sparse_core_info()
    nw = info.num_cores * info.num_subcores
    nl = info.num_lanes
    per_w = n_total // nw
    nchunk = per_w // chunk
    assert nchunk % 2 == 0

    mesh = plsc.VectorSubcoreMesh(core_axis_name="c", subcore_axis_name="s")

    @functools.partial(
        pl.kernel,
        mesh=mesh,
        out_type=jax.ShapeDtypeStruct((n_total,), jnp.float32),
        compiler_params=pltpu.CompilerParams(needs_layout_passes=False),
        scratch_types=[
            pltpu.VMEM((per_w,), jnp.int32),
            pltpu.VMEM((per_w,), jnp.int32),
            pltpu.VMEM((per_w,), jnp.int32),
            pltpu.VMEM((chunk * DIM,), jnp.float32),
            pltpu.VMEM((chunk, DIM), jnp.float32),
            pltpu.VMEM((chunk * DIM,), jnp.float32),
            pltpu.VMEM((chunk * DIM,), jnp.float32),
            pltpu.VMEM((chunk, DIM), jnp.float32),
            pltpu.VMEM((chunk * DIM,), jnp.float32),
            pltpu.VMEM((16 * 6 * 16,), jnp.float32),
            pltpu.VMEM((per_w,), jnp.float32),
            pltpu.SemaphoreType.DMA,
            pltpu.SemaphoreType.DMA,
        ],
    )
    def score_kernel(gout_hbm, rels_hbm, hpos_hbm, ridx_hbm, tpos_hbm,
                     out_hbm, idxh, idxr, idxt, hbufa, rbufa, tbufa,
                     hbufb, rbufb, tbufb, stage, scores_v, sema, semb):
        wid = lax.axis_index("s") * info.num_cores + lax.axis_index("c")
        pltpu.sync_copy(hpos_hbm.at[wid], idxh)
        pltpu.sync_copy(ridx_hbm.at[wid], idxr)
        pltpu.sync_copy(tpos_hbm.at[wid], idxt)
        lanes = lax.iota(jnp.int32, nl)
        lanes_cols = lanes * (6 * nl)

        def fire(g, hb, rb_, tb, s):
            base = g * chunk

            def dma_body(q, c2):
                qb = q * nl
                vh = idxh[pl.ds(base + qb, nl)]
                vr = idxr[pl.ds(base + qb, nl)]
                vt = idxt[pl.ds(base + qb, nl)]
                for rm in range(nl):
                    j = qb + rm
                    pltpu.async_copy(
                        gout_hbm.at[pl.ds(vh[rm] * DIM, DIM)],
                        hb.at[pl.ds(j * DIM, DIM)], s)
                    pltpu.async_copy(
                        rels_hbm.at[pl.ds(vr[rm], 1)],
                        rb_.at[pl.ds(j, 1)], s)
                    pltpu.async_copy(
                        gout_hbm.at[pl.ds(vt[rm] * DIM, DIM)],
                        tb.at[pl.ds(j * DIM, DIM)], s)
                return c2

            lax.fori_loop(0, chunk // nl, dma_body, 0)

        def drain(hb, rb_, tb, s):
            pltpu.make_async_copy(
                gout_hbm.at[pl.ds(0, chunk * DIM)], hb, s).wait()
            pltpu.make_async_copy(
                rels_hbm.at[pl.ds(0, chunk)], rb_, s).wait()
            pltpu.make_async_copy(
                gout_hbm.at[pl.ds(0, chunk * DIM)], tb, s).wait()

        def compute(g, hb, rb_, tb):
            base = g * chunk

            def rb_body(rb, carry2):
                base_r = rb * nl
                for rm in range(nl):
                    r = base_r + rm
                    h = [hb[pl.ds(r * DIM + j * nl, nl)]
                         for j in range(DIM // nl)]
                    rv = [rb_[r, pl.ds(j * nl, nl)]
                          for j in range(DIM // nl)]
                    t = [tb[pl.ds(r * DIM + j * nl, nl)]
                         for j in range(DIM // nl)]
                    prods = (
                        _tree_sum([x * x for x in h]),
                        _tree_sum([x * x for x in rv]),
                        _tree_sum([x * x for x in t]),
                        _tree_sum([x * y for x, y in zip(h, rv)]),
                        _tree_sum([x * y for x, y in zip(h, t)]),
                        _tree_sum([x * y for x, y in zip(rv, t)]),
                    )
                    for k, v in enumerate(prods):
                        plsc.store_scatter(
                            stage, [lanes_cols + (k * nl + rm)], v)

                tot = [
                    _tree_sum([stage[pl.ds(j * 6 * nl + k * nl, nl)]
                               for j in range(nl)])
                    for k in range(6)
                ]
                hh, rr, tt, hr, ht, rt = tot
                ia = _rsqrt(jnp.maximum(hh, EPS2))
                ib = _rsqrt(jnp.maximum(rr, EPS2))
                ic = _rsqrt(jnp.maximum(tt, EPS2))
                s2 = (hh * ia * ia + rr * ib * ib + tt * ic * ic
                      + 2.0 * (hr * (ia * ib) - ht * (ia * ic)
                               - rt * (ib * ic)))
                s2 = jnp.maximum(s2, 0.0)
                score = s2 * _rsqrt(jnp.maximum(s2, 1e-30))
                scores_v[pl.ds(base + base_r, nl)] = score
                return carry2

            lax.fori_loop(0, chunk // nl, rb_body, 0)

        fire(0, hbufa, rbufa, tbufa, sema)

        def pair_body(tpair, carry):
            ge = 2 * tpair
            go = ge + 1
            fire(go, hbufb, rbufb, tbufb, semb)
            drain(hbufa, rbufa, tbufa, sema)
            compute(ge, hbufa, rbufa, tbufa)

            @pl.when(tpair + 1 < nchunk // 2)
            def _():
                fire(go + 1, hbufa, rbufa, tbufa, sema)

            drain(hbufb, rbufb, tbufb, semb)
            compute(go, hbufb, rbufb, tbufb)
            return carry

        lax.fori_loop(0, nchunk // 2, pair_body, 0)
        pltpu.sync_copy(scores_v, out_hbm.at[pl.ds(wid * per_w, per_w)])

    return score_kernel, nw


def kernel(heads, rels, tails, sources, heads_bad, rels_bad, tails_bad,
           sources_bad, ents_weight, rels_weight):
    n = heads.shape[0]
    n_total = 2 * n
    n_req = 2 * n_total
    n_ents_p1 = ents_weight.shape[0]
    chunk = 128

    xk, nw, nbw, nblk, base, extra = _make_extract_kernel(n_ents_p1, n_req)
    sk, _ = _make_score_kernel(n_total, nw * nbw * SLOT, chunk)
    per_w = n_total // nw

    all_heads = jnp.concatenate([heads, heads_bad])
    all_rels = jnp.concatenate([rels, rels_bad])
    all_tails = jnp.concatenate([tails, tails_bad])

    eidx = jnp.concatenate([all_heads, all_tails])
    order = jnp.argsort(eidx).astype(jnp.int32)
    sorted_eidx = jnp.take(eidx, order).astype(jnp.int32)
    inv = jnp.zeros((n_req,), jnp.int32).at[order].set(
        jnp.arange(n_req, dtype=jnp.int32))
    blk_starts = jnp.arange(nblk + 1, dtype=jnp.int32) * BCOL
    bounds = jnp.searchsorted(sorted_eidx, blk_starts).astype(jnp.int32)
    bounds_p = jnp.concatenate(
        [bounds, jnp.full((21,), n_req, jnp.int32)])
    blkid = sorted_eidx // BCOL
    tec = jnp.where(blkid < extra * (base + 1),
                    blkid // (base + 1), (blkid - extra) // base)
    firstb = base * tec + jnp.minimum(tec, extra)
    pos_sorted = ((tec * nbw + (blkid - firstb)) * SLOT
                  + jnp.arange(n_req, dtype=jnp.int32)
                  - jnp.take(bounds, blkid))
    gp = jnp.take(pos_sorted, inv)
    gph = gp[:n_total].reshape(nw, per_w)
    gpt = gp[n_total:].reshape(nw, per_w)
    ridx = all_rels.reshape(nw, per_w)

    tail_t = ents_weight[(nblk - 1) * BCOL:].T
    gout = xk(ents_weight.T, tail_t, sorted_eidx, bounds_p)
    scores = sk(gout, rels_weight, gph, ridx, gpt)
    scores = scores.reshape(2, n)
    return (scores[0], scores[1])


# speedup vs baseline: 1.7821x; 1.7821x over previous
import functools

import jax
import jax.numpy as jnp
from jax import lax
from jax.experimental import pallas as pl
from jax.experimental.pallas import tpu as pltpu
from jax.experimental.pallas import tpu_sc as plsc

DIM = 64
BCOL = 512
SLOT = 96
EPS2 = 1e-24


def _rsqrt(x):
    i = plsc.bitcast(x, jnp.int32)
    i = jnp.int32(0x5F3759DF) - lax.shift_right_arithmetic(i, 1)
    y = plsc.bitcast(i, jnp.float32)
    for _ in range(3):
        y = y * (1.5 - 0.5 * x * y * y)
    return y


def _tree_sum(vs):
    while len(vs) > 1:
        vs = [a + b for a, b in zip(vs[::2], vs[1::2])]
    return vs[0]


@functools.lru_cache(maxsize=None)
def _make_extract_kernel(n_ents_p1: int, n_req: int):
    info = plsc.get_sparse_core_info()
    nw = info.num_cores * info.num_subcores
    nl = info.num_lanes
    nblk = -(-n_ents_p1 // BCOL)
    lastw = n_ents_p1 - (nblk - 1) * BCOL
    base = nblk // nw
    extra = nblk - base * nw
    nbw = base + (1 if extra else 0)
    assert nbw % 2 == 0
    win = 4096
    nbounds = nblk + 1 + 21

    mesh = plsc.VectorSubcoreMesh(core_axis_name="c", subcore_axis_name="s")

    @functools.partial(
        pl.kernel,
        mesh=mesh,
        out_type=jax.ShapeDtypeStruct((nw * nbw * SLOT * DIM,), jnp.float32),
        compiler_params=pltpu.CompilerParams(needs_layout_passes=False),
        scratch_types=[
            pltpu.VMEM((nbounds,), jnp.int32),
            pltpu.VMEM((win + nl,), jnp.int32),
            pltpu.VMEM((DIM, BCOL), jnp.float32),
            pltpu.VMEM((DIM, BCOL), jnp.float32),
            pltpu.VMEM((DIM, lastw), jnp.float32),
            pltpu.VMEM((SLOT * DIM,), jnp.float32),
            pltpu.VMEM((SLOT * DIM,), jnp.float32),
            pltpu.SemaphoreType.DMA,
            pltpu.SemaphoreType.DMA,
            pltpu.SemaphoreType.DMA,
            pltpu.SemaphoreType.DMA,
        ],
    )
    def extract_kernel(entst_hbm, tail_hbm, sidx_hbm, bounds_hbm, gout_hbm,
                       bounds_v, sidx_v, blka, blkb, tailv, stga, stgb,
                       ssa, ssb, swa, swb):
        wid = lax.axis_index("s") * info.num_cores + lax.axis_index("c")
        myblocks = base + jnp.where(wid < extra, 1, 0)
        first = base * wid + jnp.minimum(wid, extra)
        pltpu.sync_copy(bounds_hbm, bounds_v)
        vf = bounds_v[pl.ds(first, nl)]
        rs0 = vf[0]
        rwin = pl.multiple_of(jnp.minimum(rs0, n_req - win) & jnp.int32(~7), 8)
        pltpu.sync_copy(sidx_hbm.at[pl.ds(rwin, win)],
                        sidx_v.at[pl.ds(0, win)])
        pltpu.sync_copy(tail_hbm, tailv)
        lanes = lax.iota(jnp.int32, nl)
        lanes_d = lanes * DIM
        gbase = wid * (nbw * SLOT * DIM)

        def fire(t, blk, ss):
            b = first + t

            @pl.when(b < nblk - 1)
            def _():
                pltpu.async_copy(
                    entst_hbm.at[:, pl.ds(b * BCOL, BCOL)], blk, ss)

        def drain_stream(t, blk, ss):
            b = first + t

            @pl.when(b < nblk - 1)
            def _():
                pltpu.make_async_copy(
                    entst_hbm.at[:, pl.ds(0, BCOL)], blk, ss).wait()

        def drain_write(stg, sw):
            pltpu.make_async_copy(
                gout_hbm.at[pl.ds(0, SLOT * DIM)], stg, sw).wait()

        def process(t, blk, stg, sw):
            b = first + t
            col0 = b * BCOL
            vb = bounds_v[pl.ds(b, nl)]
            b0, b1 = vb[0], vb[1]
            ngroups = lax.shift_right_logical(b1 - b0 + (nl - 1), 4)
            zi = jnp.zeros((nl,), jnp.int32)

            def grp_from(src, width):
                def grp(gi, carry):
                    widx = b0 - rwin + gi * nl
                    ev = sidx_v[pl.ds(widx, nl)]
                    cols = jnp.clip(ev - col0, 0, width - 1)
                    for j in range(DIM):
                        g = plsc.load_gather(src, [zi + j, cols])
                        plsc.store_scatter(
                            stg, [lanes_d + (gi * nl * DIM + j)], g)
                    return carry
                return grp

            @pl.when(b < nblk - 1)
            def _():
                lax.fori_loop(0, ngroups, grp_from(blk, BCOL), 0)

            @pl.when(b == nblk - 1)
            def _():
                lax.fori_loop(0, ngroups, grp_from(tailv, lastw), 0)

            pltpu.async_copy(
                stg, gout_hbm.at[pl.ds(gbase + t * (SLOT * DIM),
                                       SLOT * DIM)], sw)

        @pl.when(0 < myblocks)
        def _():
            fire(0, blka, ssa)

        def pair_body(t2, carry):
            be = 2 * t2
            bo = be + 1

            @pl.when(bo < myblocks)
            def _():
                fire(bo, blkb, ssb)

            @pl.when(be < myblocks)
            def _():
                @pl.when(be >= 2)
                def _():
                    drain_write(stga, swa)

                drain_stream(be, blka, ssa)
                process(be, blka, stga, swa)

            @pl.when(bo + 1 < myblocks)
            def _():
                fire(bo + 1, blka, ssa)

            @pl.when(bo < myblocks)
            def _():
                @pl.when(bo >= 2)
                def _():
                    drain_write(stgb, swb)

                drain_stream(bo, blkb, ssb)
                process(bo, blkb, stgb, swb)

            return carry

        lax.fori_loop(0, nbw // 2, pair_body, 0)
        drain_write(stga, swa)
        drain_write(stgb, swb)

    return extract_kernel, nw, nbw, nblk, base, extra


@functools.lru_cache(maxsize=None)
def _make_score_kernel(n_total: int, n_gout: int, chunk: int):
    info = plsc.get_sparse_core_info()
    nw = info.num_cores * info.num_subcores
    nl = info.num_lanes
    per_w = n_total // nw
    nchunk = per_w // chunk
    assert nchunk % 2 == 0

    mesh = plsc.VectorSubcoreMesh(core_axis_name="c", subcore_axis_name="s")

    @functools.partial(
        pl.kernel,
        mesh=mesh,
        out_type=jax.ShapeDtypeStruct((n_total,), jnp.float32),
        compiler_params=pltpu.CompilerParams(needs_layout_passes=False),
        scratch_types=[
            pltpu.VMEM((per_w,), jnp.int32),
            pltpu.VMEM((per_w,), jnp.int32),
            pltpu.VMEM((per_w,), jnp.int32),
            pltpu.VMEM((chunk * DIM,), jnp.float32),
            pltpu.VMEM((chunk, DIM), jnp.float32),
            pltpu.VMEM((chunk * DIM,), jnp.float32),
            pltpu.VMEM((chunk * DIM,), jnp.float32),
            pltpu.VMEM((chunk, DIM), jnp.float32),
            pltpu.VMEM((chunk * DIM,), jnp.float32),
            pltpu.VMEM((16 * 6 * 16,), jnp.float32),
            pltpu.VMEM((per_w,), jnp.float32),
            pltpu.SemaphoreType.DMA,
            pltpu.SemaphoreType.DMA,
        ],
    )
    def score_kernel(gout_hbm, rels_hbm, hpos_hbm, ridx_hbm, tpos_hbm,
                     out_hbm, idxh, idxr, idxt, hbufa, rbufa, tbufa,
                     hbufb, rbufb, tbufb, stage, scores_v, sema, semb):
        wid = lax.axis_index("s") * info.num_cores + lax.axis_index("c")
        pltpu.sync_copy(hpos_hbm.at[wid], idxh)
        pltpu.sync_copy(ridx_hbm.at[wid], idxr)
        pltpu.sync_copy(tpos_hbm.at[wid], idxt)
        lanes = lax.iota(jnp.int32, nl)
        lanes_cols = lanes * (6 * nl)

        def fire(g, hb, rb_, tb, s):
            base = g * chunk

            def dma_body(q, c2):
                qb = q * nl
                vh = idxh[pl.ds(base + qb, nl)]
                vr = idxr[pl.ds(base + qb, nl)]
                vt = idxt[pl.ds(base + qb, nl)]
                for rm in range(nl):
                    j = qb + rm
                    pltpu.async_copy(
                        gout_hbm.at[pl.ds(vh[rm] * DIM, DIM)],
                        hb.at[pl.ds(j * DIM, DIM)], s)
                    pltpu.async_copy(
                        rels_hbm.at[pl.ds(vr[rm], 1)],
                        rb_.at[pl.ds(j, 1)], s)
                    pltpu.async_copy(
                        gout_hbm.at[pl.ds(vt[rm] * DIM, DIM)],
                        tb.at[pl.ds(j * DIM, DIM)], s)
                return c2

            lax.fori_loop(0, chunk // nl, dma_body, 0)

        def drain(hb, rb_, tb, s):
            pltpu.make_async_copy(
                gout_hbm.at[pl.ds(0, chunk * DIM)], hb, s).wait()
            pltpu.make_async_copy(
                rels_hbm.at[pl.ds(0, chunk)], rb_, s).wait()
            pltpu.make_async_copy(
                gout_hbm.at[pl.ds(0, chunk * DIM)], tb, s).wait()

        def compute(g, hb, rb_, tb):
            base = g * chunk

            def rb_body(rb, carry2):
                base_r = rb * nl
                for rm in range(nl):
                    r = base_r + rm
                    h = [hb[pl.ds(r * DIM + j * nl, nl)]
                         for j in range(DIM // nl)]
                    rv = [rb_[r, pl.ds(j * nl, nl)]
                          for j in range(DIM // nl)]
                    t = [tb[pl.ds(r * DIM + j * nl, nl)]
                         for j in range(DIM // nl)]
                    prods = (
                        _tree_sum([x * x for x in h]),
                        _tree_sum([x * x for x in rv]),
                        _tree_sum([x * x for x in t]),
                        _tree_sum([x * y for x, y in zip(h, rv)]),
                        _tree_sum([x * y for x, y in zip(h, t)]),
                        _tree_sum([x * y for x, y in zip(rv, t)]),
                    )
                    for k, v in enumerate(prods):
                        plsc.store_scatter(
                            stage, [lanes_cols + (k * nl + rm)], v)

                tot = [
                    _tree_sum([stage[pl.ds(j * 6 * nl + k * nl, nl)]
                               for j in range(nl)])
                    for k in range(6)
                ]
                hh, rr, tt, hr, ht, rt = tot
                ia = _rsqrt(jnp.maximum(hh, EPS2))
                ib = _rsqrt(jnp.maximum(rr, EPS2))
                ic = _rsqrt(jnp.maximum(tt, EPS2))
                s2 = (hh * ia * ia + rr * ib * ib + tt * ic * ic
                      + 2.0 * (hr * (ia * ib) - ht * (ia * ic)
                               - rt * (ib * ic)))
                s2 = jnp.maximum(s2, 0.0)
                score = s2 * _rsqrt(jnp.maximum(s2, 1e-30))
                scores_v[pl.ds(base + base_r, nl)] = score
                return carry2

            lax.fori_loop(0, chunk // nl, rb_body, 0)

        fire(0, hbufa, rbufa, tbufa, sema)

        def pair_body(tpair, carry):
            ge = 2 * tpair
            go = ge + 1
            fire(go, hbufb, rbufb, tbufb, semb)
            drain(hbufa, rbufa, tbufa, sema)
            compute(ge, hbufa, rbufa, tbufa)

            @pl.when(tpair + 1 < nchunk // 2)
            def _():
                fire(go + 1, hbufa, rbufa, tbufa, sema)

            drain(hbufb, rbufb, tbufb, semb)
            compute(go, hbufb, rbufb, tbufb)
            return carry

        lax.fori_loop(0, nchunk // 2, pair_body, 0)
        pltpu.sync_copy(scores_v, out_hbm.at[pl.ds(wid * per_w, per_w)])

    return score_kernel, nw


def kernel(heads, rels, tails, sources, heads_bad, rels_bad, tails_bad,
           sources_bad, ents_weight, rels_weight):
    n = heads.shape[0]
    n_total = 2 * n
    n_req = 2 * n_total
    n_ents_p1 = ents_weight.shape[0]
    chunk = 128

    xk, nw, nbw, nblk, base, extra = _make_extract_kernel(n_ents_p1, n_req)
    sk, _ = _make_score_kernel(n_total, nw * nbw * SLOT, chunk)
    per_w = n_total // nw

    all_heads = jnp.concatenate([heads, heads_bad])
    all_rels = jnp.concatenate([rels, rels_bad])
    all_tails = jnp.concatenate([tails, tails_bad])

    eidx = jnp.concatenate([all_heads, all_tails])
    s_arange = jnp.arange(n_req, dtype=jnp.int32)
    sorted_eidx, order = lax.sort([eidx, s_arange], num_keys=1)
    blk_starts = jnp.arange(nblk + 1, dtype=jnp.int32) * BCOL
    bounds = jnp.searchsorted(
        sorted_eidx, blk_starts, method="sort").astype(jnp.int32)
    bounds_p = jnp.concatenate(
        [bounds, jnp.full((21,), n_req, jnp.int32)])
    blkid = sorted_eidx // BCOL
    tec = jnp.where(blkid < extra * (base + 1),
                    blkid // (base + 1), (blkid - extra) // base)
    firstb = base * tec + jnp.minimum(tec, extra)
    isfirst = jnp.concatenate(
        [jnp.array([True]), blkid[1:] != blkid[:-1]])
    b0 = lax.cummax(jnp.where(isfirst, s_arange, 0))
    pos_sorted = ((tec * nbw + (blkid - firstb)) * SLOT
                  + s_arange - b0)
    _, gp = lax.sort([order, pos_sorted], num_keys=1)
    gph = gp[:n_total].reshape(nw, per_w)
    gpt = gp[n_total:].reshape(nw, per_w)
    ridx = all_rels.reshape(nw, per_w)

    tail_t = ents_weight[(nblk - 1) * BCOL:].T
    gout = xk(ents_weight.T, tail_t, sorted_eidx, bounds_p)
    scores = sk(gout, rels_weight, gph, ridx, gpt)
    scores = scores.reshape(2, n)
    return (scores[0], scores[1])


# speedup vs baseline: 4.0150x; 2.2529x over previous
import functools

import jax
import jax.numpy as jnp
from jax import lax
from jax.experimental import pallas as pl
from jax.experimental.pallas import tpu as pltpu
from jax.experimental.pallas import tpu_sc as plsc

DIM = 64
BCOL = 512
SLOT = 96
EPS2 = 1e-24


def _rsqrt(x):
    i = plsc.bitcast(x, jnp.int32)
    i = jnp.int32(0x5F3759DF) - lax.shift_right_arithmetic(i, 1)
    y = plsc.bitcast(i, jnp.float32)
    for _ in range(3):
        y = y * (1.5 - 0.5 * x * y * y)
    return y


def _tree_sum(vs):
    while len(vs) > 1:
        vs = [a + b for a, b in zip(vs[::2], vs[1::2])]
    return vs[0]


@functools.lru_cache(maxsize=None)
def _make_extract_kernel(n_ents_p1: int, n_req: int):
    info = plsc.get_sparse_core_info()
    nw = info.num_cores * info.num_subcores
    nl = info.num_lanes
    nblk = -(-n_ents_p1 // BCOL)
    lastw = n_ents_p1 - (nblk - 1) * BCOL
    base = nblk // nw
    extra = nblk - base * nw
    nbw = base + (1 if extra else 0)
    assert nbw % 2 == 0
    win = 4096
    wbits = 13

    mesh = plsc.VectorSubcoreMesh(core_axis_name="c", subcore_axis_name="s")

    @functools.partial(
        pl.kernel,
        mesh=mesh,
        out_type=jax.ShapeDtypeStruct((nw * nbw * SLOT * DIM,), jnp.float32),
        compiler_params=pltpu.CompilerParams(needs_layout_passes=False),
        scratch_types=[
            pltpu.VMEM((nw + nl,), jnp.int32),
            pltpu.VMEM((win + nl,), jnp.int32),
            pltpu.VMEM((DIM, BCOL), jnp.float32),
            pltpu.VMEM((DIM, BCOL), jnp.float32),
            pltpu.VMEM((DIM, lastw), jnp.float32),
            pltpu.VMEM((SLOT * DIM,), jnp.float32),
            pltpu.VMEM((SLOT * DIM,), jnp.float32),
            pltpu.SemaphoreType.DMA,
            pltpu.SemaphoreType.DMA,
            pltpu.SemaphoreType.DMA,
            pltpu.SemaphoreType.DMA,
        ],
    )
    def extract_kernel(entst_hbm, tail_hbm, sidx_hbm, rstart_hbm, gout_hbm,
                       rstart_v, sidx_v, blka, blkb, tailv, stga, stgb,
                       ssa, ssb, swa, swb):
        wid = lax.axis_index("s") * info.num_cores + lax.axis_index("c")
        myblocks = base + jnp.where(wid < extra, 1, 0)
        first = base * wid + jnp.minimum(wid, extra)
        pltpu.sync_copy(rstart_hbm, rstart_v)
        vf = rstart_v[pl.ds(wid, nl)]
        rs0 = vf[0]
        rwin = pl.multiple_of(jnp.minimum(rs0, n_req - win) & jnp.int32(~7), 8)
        pltpu.sync_copy(sidx_hbm.at[pl.ds(rwin, win)],
                        sidx_v.at[pl.ds(0, win)])
        pltpu.sync_copy(tail_hbm, tailv)
        lanes = lax.iota(jnp.int32, nl)
        lanes_d = lanes * DIM
        gbase = wid * (nbw * SLOT * DIM)

        def fire(t, blk, ss):
            b = first + t

            @pl.when(b < nblk - 1)
            def _():
                pltpu.async_copy(
                    entst_hbm.at[:, pl.ds(b * BCOL, BCOL)], blk, ss)

        def drain_stream(t, blk, ss):
            b = first + t

            @pl.when(b < nblk - 1)
            def _():
                pltpu.make_async_copy(
                    entst_hbm.at[:, pl.ds(0, BCOL)], blk, ss).wait()

        def drain_write(stg, sw):
            pltpu.make_async_copy(
                gout_hbm.at[pl.ds(0, SLOT * DIM)], stg, sw).wait()

        def bsearch(tgt):
            def step(i, lohi):
                lo, hi = lohi
                mid = lax.shift_right_logical(lo + hi, 1)
                v = sidx_v[pl.ds(mid, nl)]
                go = v[0] < tgt
                return (jnp.where(go, mid + 1, lo), jnp.where(go, hi, mid))

            lo, _ = lax.fori_loop(0, wbits, step, (jnp.int32(0),
                                                   jnp.int32(win)))
            return lo

        def process(t, blk, stg, sw):
            b = first + t
            col0 = b * BCOL
            wlo = bsearch(col0)
            whi = bsearch(col0 + BCOL)
            ngroups = lax.shift_right_logical(whi - wlo + (nl - 1), 4)
            zi = jnp.zeros((nl,), jnp.int32)

            def grp_from(src, width):
                def grp(gi, carry):
                    widx = wlo + gi * nl
                    ev = sidx_v[pl.ds(widx, nl)]
                    cols = jnp.clip(ev - col0, 0, width - 1)
                    for j in range(DIM):
                        g = plsc.load_gather(src, [zi + j, cols])
                        plsc.store_scatter(
                            stg, [lanes_d + (gi * nl * DIM + j)], g)
                    return carry
                return grp

            @pl.when(b < nblk - 1)
            def _():
                lax.fori_loop(0, ngroups, grp_from(blk, BCOL), 0)

            @pl.when(b == nblk - 1)
            def _():
                lax.fori_loop(0, ngroups, grp_from(tailv, lastw), 0)

            pltpu.async_copy(
                stg, gout_hbm.at[pl.ds(gbase + t * (SLOT * DIM),
                                       SLOT * DIM)], sw)

        @pl.when(0 < myblocks)
        def _():
            fire(0, blka, ssa)

        def pair_body(t2, carry):
            be = 2 * t2
            bo = be + 1

            @pl.when(bo < myblocks)
            def _():
                fire(bo, blkb, ssb)

            @pl.when(be < myblocks)
            def _():
                @pl.when(be >= 2)
                def _():
                    drain_write(stga, swa)

                drain_stream(be, blka, ssa)
                process(be, blka, stga, swa)

            @pl.when(bo + 1 < myblocks)
            def _():
                fire(bo + 1, blka, ssa)

            @pl.when(bo < myblocks)
            def _():
                @pl.when(bo >= 2)
                def _():
                    drain_write(stgb, swb)

                drain_stream(bo, blkb, ssb)
                process(bo, blkb, stgb, swb)

            return carry

        lax.fori_loop(0, nbw // 2, pair_body, 0)
        drain_write(stga, swa)
        drain_write(stgb, swb)

    return extract_kernel, nw, nbw, nblk, base, extra


@functools.lru_cache(maxsize=None)
def _make_score_kernel(n_total: int, n_gout: int, chunk: int):
    info = plsc.get_sparse_core_info()
    nw = info.num_cores * info.num_subcores
    nl = info.num_lanes
    per_w = n_total // nw
    nchunk = per_w // chunk
    assert nchunk % 2 == 0

    mesh = plsc.VectorSubcoreMesh(core_axis_name="c", subcore_axis_name="s")

    @functools.partial(
        pl.kernel,
        mesh=mesh,
        out_type=jax.ShapeDtypeStruct((n_total,), jnp.float32),
        compiler_params=pltpu.CompilerParams(needs_layout_passes=False),
        scratch_types=[
            pltpu.VMEM((per_w,), jnp.int32),
            pltpu.VMEM((per_w,), jnp.int32),
            pltpu.VMEM((per_w,), jnp.int32),
            pltpu.VMEM((chunk * DIM,), jnp.float32),
            pltpu.VMEM((chunk, DIM), jnp.float32),
            pltpu.VMEM((chunk * DIM,), jnp.float32),
            pltpu.VMEM((chunk * DIM,), jnp.float32),
            pltpu.VMEM((chunk, DIM), jnp.float32),
            pltpu.VMEM((chunk * DIM,), jnp.float32),
            pltpu.VMEM((16 * 6 * 16,), jnp.float32),
            pltpu.VMEM((per_w,), jnp.float32),
            pltpu.SemaphoreType.DMA,
            pltpu.SemaphoreType.DMA,
        ],
    )
    def score_kernel(gout_hbm, rels_hbm, hpos_hbm, ridx_hbm, tpos_hbm,
                     out_hbm, idxh, idxr, idxt, hbufa, rbufa, tbufa,
                     hbufb, rbufb, tbufb, stage, scores_v, sema, semb):
        wid = lax.axis_index("s") * info.num_cores + lax.axis_index("c")
        pltpu.sync_copy(hpos_hbm.at[wid], idxh)
        pltpu.sync_copy(ridx_hbm.at[wid], idxr)
        pltpu.sync_copy(tpos_hbm.at[wid], idxt)
        lanes = lax.iota(jnp.int32, nl)
        lanes_cols = lanes * (6 * nl)

        def fire(g, hb, rb_, tb, s):
            base = g * chunk

            def dma_body(q, c2):
                qb = q * nl
                vh = idxh[pl.ds(base + qb, nl)]
                vr = idxr[pl.ds(base + qb, nl)]
                vt = idxt[pl.ds(base + qb, nl)]
                for rm in range(nl):
                    j = qb + rm
                    pltpu.async_copy(
                        gout_hbm.at[pl.ds(vh[rm] * DIM, DIM)],
                        hb.at[pl.ds(j * DIM, DIM)], s)
                    pltpu.async_copy(
                        rels_hbm.at[pl.ds(vr[rm], 1)],
                        rb_.at[pl.ds(j, 1)], s)
                    pltpu.async_copy(
                        gout_hbm.at[pl.ds(vt[rm] * DIM, DIM)],
                        tb.at[pl.ds(j * DIM, DIM)], s)
                return c2

            lax.fori_loop(0, chunk // nl, dma_body, 0)

        def drain(hb, rb_, tb, s):
            pltpu.make_async_copy(
                gout_hbm.at[pl.ds(0, chunk * DIM)], hb, s).wait()
            pltpu.make_async_copy(
                rels_hbm.at[pl.ds(0, chunk)], rb_, s).wait()
            pltpu.make_async_copy(
                gout_hbm.at[pl.ds(0, chunk * DIM)], tb, s).wait()

        def compute(g, hb, rb_, tb):
            base = g * chunk

            def rb_body(rb, carry2):
                base_r = rb * nl
                for rm in range(nl):
                    r = base_r + rm
                    h = [hb[pl.ds(r * DIM + j * nl, nl)]
                         for j in range(DIM // nl)]
                    rv = [rb_[r, pl.ds(j * nl, nl)]
                          for j in range(DIM // nl)]
                    t = [tb[pl.ds(r * DIM + j * nl, nl)]
                         for j in range(DIM // nl)]
                    prods = (
                        _tree_sum([x * x for x in h]),
                        _tree_sum([x * x for x in rv]),
                        _tree_sum([x * x for x in t]),
                        _tree_sum([x * y for x, y in zip(h, rv)]),
                        _tree_sum([x * y for x, y in zip(h, t)]),
                        _tree_sum([x * y for x, y in zip(rv, t)]),
                    )
                    for k, v in enumerate(prods):
                        plsc.store_scatter(
                            stage, [lanes_cols + (k * nl + rm)], v)

                tot = [
                    _tree_sum([stage[pl.ds(j * 6 * nl + k * nl, nl)]
                               for j in range(nl)])
                    for k in range(6)
                ]
                hh, rr, tt, hr, ht, rt = tot
                ia = _rsqrt(jnp.maximum(hh, EPS2))
                ib = _rsqrt(jnp.maximum(rr, EPS2))
                ic = _rsqrt(jnp.maximum(tt, EPS2))
                s2 = (hh * ia * ia + rr * ib * ib + tt * ic * ic
                      + 2.0 * (hr * (ia * ib) - ht * (ia * ic)
                               - rt * (ib * ic)))
                s2 = jnp.maximum(s2, 0.0)
                score = s2 * _rsqrt(jnp.maximum(s2, 1e-30))
                scores_v[pl.ds(base + base_r, nl)] = score
                return carry2

            lax.fori_loop(0, chunk // nl, rb_body, 0)

        fire(0, hbufa, rbufa, tbufa, sema)

        def pair_body(tpair, carry):
            ge = 2 * tpair
            go = ge + 1
            fire(go, hbufb, rbufb, tbufb, semb)
            drain(hbufa, rbufa, tbufa, sema)
            compute(ge, hbufa, rbufa, tbufa)

            @pl.when(tpair + 1 < nchunk // 2)
            def _():
                fire(go + 1, hbufa, rbufa, tbufa, sema)

            drain(hbufb, rbufb, tbufb, semb)
            compute(go, hbufb, rbufb, tbufb)
            return carry

        lax.fori_loop(0, nchunk // 2, pair_body, 0)
        pltpu.sync_copy(scores_v, out_hbm.at[pl.ds(wid * per_w, per_w)])

    return score_kernel, nw


def kernel(heads, rels, tails, sources, heads_bad, rels_bad, tails_bad,
           sources_bad, ents_weight, rels_weight):
    n = heads.shape[0]
    n_total = 2 * n
    n_req = 2 * n_total
    n_ents_p1 = ents_weight.shape[0]
    chunk = 128

    xk, nw, nbw, nblk, base, extra = _make_extract_kernel(n_ents_p1, n_req)
    sk, _ = _make_score_kernel(n_total, nw * nbw * SLOT, chunk)
    per_w = n_total // nw

    all_heads = jnp.concatenate([heads, heads_bad])
    all_rels = jnp.concatenate([rels, rels_bad])
    all_tails = jnp.concatenate([tails, tails_bad])

    eidx = jnp.concatenate([all_heads, all_tails])
    s_arange = jnp.arange(n_req, dtype=jnp.int32)
    sorted_eidx, order = lax.sort([eidx, s_arange], num_keys=1,
                                  is_stable=False)
    blkid = sorted_eidx // BCOL
    tec = jnp.where(blkid < extra * (base + 1),
                    blkid // (base + 1), (blkid - extra) // base)
    firstb = base * tec + jnp.minimum(tec, extra)
    isfirst = jnp.concatenate(
        [jnp.array([True]), blkid[1:] != blkid[:-1]])
    b0 = lax.cummax(jnp.where(isfirst, s_arange, 0))
    pos_sorted = ((tec * nbw + (blkid - firstb)) * SLOT
                  + s_arange - b0)
    _, gp = lax.sort([order, pos_sorted], num_keys=1,
                     is_stable=False)
    isfw = jnp.concatenate([jnp.array([True]), tec[1:] != tec[:-1]])
    wkey = jnp.where(isfw, tec, jnp.int32(1) << 30)
    _, wstart = lax.sort([wkey, s_arange], num_keys=1, is_stable=False)
    rstart_p = jnp.concatenate(
        [wstart[:nw], jnp.full((16,), n_req, jnp.int32)])
    gph = gp[:n_total].reshape(nw, per_w)
    gpt = gp[n_total:].reshape(nw, per_w)
    ridx = all_rels.reshape(nw, per_w)

    tail_t = ents_weight[(nblk - 1) * BCOL:].T
    gout = xk(ents_weight.T, tail_t, sorted_eidx, rstart_p)
    scores = sk(gout, rels_weight, gph, ridx, gpt)
    scores = scores.reshape(2, n)
    return (scores[0], scores[1])
